# baseline (speedup 1.0000x reference)
import functools

import jax
import jax.numpy as jnp
from jax import lax
from jax.experimental import pallas as pl
from jax.experimental.pallas import tpu as pltpu
from jax.experimental.pallas import tpu_sc as plsc

_V = 49408
_D = 768
_T = 77
_B = 128

_NC, _NS, _L = 2, 16, 16
_NW = _NC * _NS
_FLAT = _B * _T
_PER_W = 312
_PAD = _PER_W * _NW
_CHUNK = 24
_NCHUNK = _PER_W // _CHUNK
_KV = _D // _L

@functools.cache
def _build_embed_sc():
    mesh = plsc.VectorSubcoreMesh(
        core_axis_name="c", subcore_axis_name="s", num_cores=_NC, num_subcores=_NS
    )

    @functools.partial(
        pl.kernel,
        out_type=jax.ShapeDtypeStruct((_PAD, _D), jnp.float32),
        mesh=mesh,
        scratch_types=[
            pltpu.VMEM((_PER_W,), jnp.int32),
            pltpu.VMEM((_CHUNK, _D), jnp.float32),
            pltpu.VMEM((_T, _D), jnp.float32),
            pltpu.SemaphoreType.DMA,
        ],
    )
    def _embed_sc(table_hbm, idx_hbm, pos_hbm, out_hbm, idx_v, rows_v, pos_v, sem):
        wid = lax.axis_index("s") * _NC + lax.axis_index("c")
        base = wid * _PER_W
        pltpu.sync_copy(idx_hbm.at[pl.ds(base, _PER_W)], idx_v)
        pltpu.sync_copy(pos_hbm, pos_v)
        for c in range(_NCHUNK):
            pltpu.async_copy(
                table_hbm.at[idx_v.at[pl.ds(c * _CHUNK, _CHUNK)]], rows_v, sem
            ).wait()

            def add_row(r, carry, c=c):
                t = lax.rem(base + c * _CHUNK + r, _T)
                for k in range(_KV):
                    sl = pl.ds(k * _L, _L)
                    rows_v[r, sl] = rows_v[r, sl] + pos_v[t, sl]
                return carry

            lax.fori_loop(0, _CHUNK, add_row, 0)
            pltpu.sync_copy(rows_v, out_hbm.at[pl.ds(base + c * _CHUNK, _CHUNK)])

    return _embed_sc


def kernel(tokens, token_embedding, position_embedding):
    idx = tokens.reshape(-1).astype(jnp.int32)
    idx = jnp.concatenate([idx, jnp.zeros((_PAD - _FLAT,), jnp.int32)])
    out = _build_embed_sc()(token_embedding, idx, position_embedding)
    return out[:_FLAT].reshape(_B, _T, _D)


# speedup vs baseline: 1.2936x; 1.2936x over previous
import functools

import jax
import jax.numpy as jnp
from jax import lax
from jax.experimental import pallas as pl
from jax.experimental.pallas import tpu as pltpu
from jax.experimental.pallas import tpu_sc as plsc

_V = 49408
_D = 768
_T = 77
_B = 128

_NC, _NS, _L = 2, 16, 16
_NW = _NC * _NS
_FLAT = _B * _T
_PER_W = 312
_PAD = _PER_W * _NW
_CHUNK = 24
_NCHUNK = _PER_W // _CHUNK
_KV = _D // _L

_NBUF = 3


@functools.cache
def _build_embed_sc():
    mesh = plsc.VectorSubcoreMesh(
        core_axis_name="c", subcore_axis_name="s", num_cores=_NC, num_subcores=_NS
    )

    @functools.partial(
        pl.kernel,
        out_type=jax.ShapeDtypeStruct((_PAD, _D), jnp.float32),
        mesh=mesh,
        scratch_types=[
            pltpu.VMEM((_PER_W,), jnp.int32),
            *[pltpu.VMEM((_CHUNK, _D), jnp.float32) for _ in range(_NBUF)],
            pltpu.VMEM((_T, _D), jnp.float32),
            *[pltpu.SemaphoreType.DMA for _ in range(2 * _NBUF + 1)],
        ],
    )
    def _embed_sc(
        table_hbm, idx_hbm, pos_hbm, out_hbm,
        idx_v, r0, r1, r2, pos_v, g0, g1, g2, o0, o1, o2, psem,
    ):
        rows = (r0, r1, r2)
        gsem = (g0, g1, g2)
        osem = (o0, o1, o2)
        wid = lax.axis_index("s") * _NC + lax.axis_index("c")
        base = wid * _PER_W
        pltpu.sync_copy(idx_hbm.at[pl.ds(base, _PER_W)], idx_v)
        pos_cp = pltpu.async_copy(pos_hbm, pos_v, psem)

        gather = {}
        out = {}

        def start_gather(c):
            b = c % _NBUF
            gather[c] = pltpu.async_copy(
                table_hbm.at[idx_v.at[pl.ds(c * _CHUNK, _CHUNK)]], rows[b], gsem[b]
            )

        start_gather(0)
        start_gather(1)
        pos_cp.wait()
        for c in range(_NCHUNK):
            b = c % _NBUF
            gather[c].wait()

            def add_row(r, carry, c=c, b=b):
                t = lax.rem(base + c * _CHUNK + r, _T)
                for k in range(_KV):
                    sl = pl.ds(k * _L, _L)
                    plsc.addupdate(rows[b].at[r, sl], pos_v[t, sl])
                return carry

            lax.fori_loop(0, _CHUNK, add_row, 0)
            out[c] = pltpu.async_copy(
                rows[b], out_hbm.at[pl.ds(base + c * _CHUNK, _CHUNK)], osem[b]
            )
            if c + 2 < _NCHUNK:
                if c >= 1:
                    out[c - 1].wait()
                start_gather(c + 2)
        out[_NCHUNK - 2].wait()
        out[_NCHUNK - 1].wait()

    return _embed_sc


def kernel(tokens, token_embedding, position_embedding):
    idx = tokens.reshape(-1).astype(jnp.int32)
    idx = jnp.concatenate([idx, jnp.zeros((_PAD - _FLAT,), jnp.int32)])
    out = _build_embed_sc()(token_embedding, idx, position_embedding)
    return out[:_FLAT].reshape(_B, _T, _D)


# speedup vs baseline: 1.4810x; 1.1449x over previous
import functools

import jax
import jax.numpy as jnp
from jax import lax
from jax.experimental import pallas as pl
from jax.experimental.pallas import tpu as pltpu
from jax.experimental.pallas import tpu_sc as plsc

_V = 49408
_D = 768
_T = 77
_B = 128

_NC, _NS, _L = 2, 16, 16
_NW = _NC * _NS
_FLAT = _B * _T
_PER_W = 312
_LAST = _FLAT - _PER_W
_CHUNK = 24
_NCHUNK = _PER_W // _CHUNK
_IDX_ROWS = _NCHUNK + 1
_KV = _D // _L
_NBUF = 3


@functools.cache
def _build_embed_sc():
    mesh = plsc.VectorSubcoreMesh(
        core_axis_name="c", subcore_axis_name="s", num_cores=_NC, num_subcores=_NS
    )

    @functools.partial(
        pl.kernel,
        out_type=jax.ShapeDtypeStruct((_FLAT, _D), jnp.float32),
        mesh=mesh,
        scratch_types=[
            pltpu.VMEM((_PER_W,), jnp.int32),
            *[pltpu.VMEM((_CHUNK, _D), jnp.float32) for _ in range(_NBUF)],
            pltpu.VMEM((_T, _D), jnp.float32),
            *[pltpu.SemaphoreType.DMA for _ in range(2 * _NBUF + 1)],
        ],
    )
    def _embed_sc(
        table_hbm, idx_hbm, pos_hbm, out_hbm,
        idx_v, r0, r1, r2, pos_v, g0, g1, g2, o0, o1, o2, psem,
    ):
        rows = (r0, r1, r2)
        gsem = (g0, g1, g2)
        osem = (o0, o1, o2)
        wid = lax.axis_index("s") * _NC + lax.axis_index("c")
        base = lax.min(wid * _PER_W, _LAST)
        pltpu.sync_copy(idx_hbm.at[pl.ds(base, _PER_W)], idx_v)
        pos_cp = pltpu.async_copy(pos_hbm, pos_v, psem)

        gather = {}
        out = {}

        def start_gather(c):
            b = c % _NBUF
            gather[c] = pltpu.async_copy(
                table_hbm.at[idx_v.at[pl.ds(c * _CHUNK, _CHUNK)]], rows[b], gsem[b]
            )

        start_gather(0)
        start_gather(1)
        pos_cp.wait()
        for c in range(_NCHUNK):
            b = c % _NBUF
            gather[c].wait()

            def add_row(r, carry, c=c, b=b):
                t = lax.rem(base + c * _CHUNK + r, _T)
                for k in range(_KV):
                    sl = pl.ds(k * _L, _L)
                    plsc.addupdate(rows[b].at[r, sl], pos_v[t, sl])
                return carry

            lax.fori_loop(0, _CHUNK, add_row, 0)
            out[c] = pltpu.async_copy(
                rows[b], out_hbm.at[pl.ds(base + c * _CHUNK, _CHUNK)], osem[b]
            )
            if c + 2 < _NCHUNK:
                if c >= 1:
                    out[c - 1].wait()
                start_gather(c + 2)
        out[_NCHUNK - 2].wait()
        out[_NCHUNK - 1].wait()

    return _embed_sc


def kernel(tokens, token_embedding, position_embedding):
    idx = tokens.reshape(_FLAT).astype(jnp.int32)
    out = _build_embed_sc()(token_embedding, idx, position_embedding)
    return out.reshape(_B, _T, _D)


# speedup vs baseline: 2.3192x; 1.5660x over previous
import functools

import jax
import jax.numpy as jnp
from jax import lax
from jax.experimental import pallas as pl
from jax.experimental.pallas import tpu as pltpu
from jax.experimental.pallas import tpu_sc as plsc

_V = 49408
_D = 768
_T = 77
_B = 128

_NC, _NS, _L = 2, 16, 16
_NW = _NC * _NS
_FLAT = _B * _T
_PER_W = 312
_LAST = _FLAT - _PER_W
_CHUNK = 24
_NCHUNK = _PER_W // _CHUNK
_KV = _D // _L
_NBUF = 3


@functools.cache
def _build_embed_sc():
    mesh = plsc.VectorSubcoreMesh(
        core_axis_name="c", subcore_axis_name="s", num_cores=_NC, num_subcores=_NS
    )

    @functools.partial(
        pl.kernel,
        out_type=jax.ShapeDtypeStruct((_FLAT, _D), jnp.float32),
        mesh=mesh,
        scratch_types=[
            pltpu.VMEM((_PER_W,), jnp.int32),
            *[pltpu.VMEM((_CHUNK, _D), jnp.float32) for _ in range(_NBUF)],
            pltpu.VMEM((_T, _D), jnp.float32),
            *[pltpu.SemaphoreType.DMA for _ in range(2 * _NBUF + 1)],
        ],
    )
    def _embed_sc(
        table_hbm, idx_hbm, pos_hbm, out_hbm,
        idx_v, r0, r1, r2, pos_v, g0, g1, g2, o0, o1, o2, psem,
    ):
        rows = (r0, r1, r2)
        gsem = (g0, g1, g2)
        osem = (o0, o1, o2)
        wid = lax.axis_index("s") * _NC + lax.axis_index("c")
        base = lax.min(wid * _PER_W, _LAST)
        pltpu.sync_copy(idx_hbm.at[pl.ds(base, _PER_W)], idx_v)
        pos_cp = pltpu.async_copy(pos_hbm, pos_v, psem)

        gather = {}
        out = {}

        def start_gather(c):
            b = c % _NBUF
            gather[c] = pltpu.async_copy(
                table_hbm.at[idx_v.at[pl.ds(c * _CHUNK, _CHUNK)]], rows[b], gsem[b]
            )

        start_gather(0)
        start_gather(1)
        pos_cp.wait()
        for c in range(_NCHUNK):
            b = c % _NBUF
            gather[c].wait()

            def add_row(r, carry, c=c, b=b):
                t = lax.shift_right_logical(base + c * _CHUNK + r, 7)
                for k in range(_KV):
                    sl = pl.ds(k * _L, _L)
                    plsc.addupdate(rows[b].at[r, sl], pos_v[t, sl])
                return carry

            lax.fori_loop(0, _CHUNK, add_row, 0)
            out[c] = pltpu.async_copy(
                rows[b], out_hbm.at[pl.ds(base + c * _CHUNK, _CHUNK)], osem[b]
            )
            if c + 2 < _NCHUNK:
                if c >= 1:
                    out[c - 1].wait()
                start_gather(c + 2)
        out[_NCHUNK - 2].wait()
        out[_NCHUNK - 1].wait()

    return _embed_sc


def kernel(tokens, token_embedding, position_embedding):
    idx = tokens.T.reshape(_FLAT).astype(jnp.int32)
    out = _build_embed_sc()(token_embedding, idx, position_embedding)
    return out.reshape(_T, _B, _D).transpose(1, 0, 2)
